# Initial kernel scaffold; baseline (speedup 1.0000x reference)
import jax
import jax.numpy as jnp
from jax.experimental import pallas as pl


def kernel(q, edge_index, edge_attr, W_r, W_i, W_j, W_k, b):
    raise NotImplementedError("write your pallas kernel here")


# speedup vs baseline: 68.0464x; 68.0464x over previous
import functools

import jax
import jax.numpy as jnp
from jax import lax
from jax.experimental import pallas as pl
from jax.experimental.pallas import tpu as pltpu
from jax.experimental.pallas import tpu_sc as plsc

N = 10000
E = 640000
D = 32
D4 = 4 * D

NC = 2
NS = 16
CHUNK = 80
EDGES_PER_CORE = E // NC
EDGES_PER_TILE = EDGES_PER_CORE // NS
N_CHUNKS = EDGES_PER_TILE // CHUNK
NPAD = 10240
ROWS_PER_TILE = NPAD // NS


def _sc_aggregate(q_flat, ea_flat, src, dst, zeros):
    mesh = plsc.VectorSubcoreMesh(core_axis_name="c", subcore_axis_name="s")

    @functools.partial(
        pl.kernel,
        out_type=jax.ShapeDtypeStruct((NC, NPAD, D4), jnp.float32),
        mesh=mesh,
        scratch_types=[
            pltpu.VMEM_SHARED((NPAD, D4), jnp.float32),
            pltpu.VMEM((CHUNK, D4), jnp.float32),
            pltpu.VMEM((CHUNK, D4), jnp.float32),
            pltpu.VMEM((CHUNK,), jnp.int32),
            pltpu.VMEM((CHUNK,), jnp.int32),
            pltpu.SemaphoreType.DMA,
            pltpu.SemaphoreType.DMA,
        ],
    )
    def k(q_hbm, ea_hbm, src_hbm, dst_hbm, z_hbm, out_hbm,
          accum, ea_v, q_v, src_v, dst_v, sem_e, sem_q):
        c = lax.axis_index("c")
        s = lax.axis_index("s")
        pltpu.sync_copy(z_hbm, accum.at[pl.ds(s * ROWS_PER_TILE, ROWS_PER_TILE)])
        plsc.subcore_barrier()

        base0 = c * EDGES_PER_CORE + s * EDGES_PER_TILE

        def body(i, carry):
            base = base0 + i * CHUNK
            pltpu.sync_copy(src_hbm.at[pl.ds(base, CHUNK)], src_v)
            pltpu.sync_copy(dst_hbm.at[pl.ds(base, CHUNK)], dst_v)
            cp_e = pltpu.async_copy(ea_hbm.at[pl.ds(base, CHUNK)], ea_v, sem_e)
            cp_q = pltpu.async_copy(q_hbm.at[src_v], q_v, sem_q)
            cp_e.wait()
            pltpu.sync_copy(ea_v, accum.at[dst_v], add=True)
            cp_q.wait()
            pltpu.sync_copy(q_v, accum.at[dst_v], add=True)
            return carry

        lax.fori_loop(0, N_CHUNKS, body, 0)
        plsc.subcore_barrier()
        pltpu.sync_copy(accum.at[pl.ds(s * ROWS_PER_TILE, ROWS_PER_TILE)],
                        out_hbm.at[c, pl.ds(s * ROWS_PER_TILE, ROWS_PER_TILE)])

    return k(q_flat, ea_flat, src, dst, zeros)


def _tc_transform(p, q_flat, h, b_flat):
    blk = 2000

    def body(p_ref, q_ref, h_ref, b_ref, o_ref):
        acc = p_ref[0] + p_ref[1]
        o_ref[...] = (jnp.dot(acc, h_ref[...], preferred_element_type=jnp.float32)
                      + b_ref[...] + q_ref[...])

    return pl.pallas_call(
        body,
        grid=(N // blk,),
        in_specs=[
            pl.BlockSpec((NC, blk, D4), lambda i: (0, i, 0)),
            pl.BlockSpec((blk, D4), lambda i: (i, 0)),
            pl.BlockSpec((D4, D4), lambda i: (0, 0)),
            pl.BlockSpec((1, D4), lambda i: (0, 0)),
        ],
        out_specs=pl.BlockSpec((blk, D4), lambda i: (i, 0)),
        out_shape=jax.ShapeDtypeStruct((N, D4), jnp.float32),
    )(p, q_flat, h, b_flat)


def kernel(q, edge_index, edge_attr, W_r, W_i, W_j, W_k, b):
    q_flat = q.reshape(N, D4)
    ea_flat = edge_attr.reshape(E, D4)
    src = edge_index[0].astype(jnp.int32)
    dst = edge_index[1].astype(jnp.int32)
    zeros = jnp.zeros((ROWS_PER_TILE, D4), jnp.float32)

    p = _sc_aggregate(q_flat, ea_flat, src, dst, zeros)

    h = jnp.concatenate([
        jnp.concatenate([W_r, W_i, W_j, W_k], axis=1),
        jnp.concatenate([-W_i, W_r, -W_k, W_j], axis=1),
        jnp.concatenate([-W_j, W_k, W_r, -W_i], axis=1),
        jnp.concatenate([-W_k, -W_j, W_i, W_r], axis=1),
    ], axis=0)
    b_flat = b.reshape(1, D4)

    out = _tc_transform(p, q_flat, h, b_flat)
    return out.reshape(N, 4, D)


# speedup vs baseline: 94.5968x; 1.3902x over previous
import functools

import jax
import jax.numpy as jnp
from jax import lax
from jax.experimental import pallas as pl
from jax.experimental.pallas import tpu as pltpu
from jax.experimental.pallas import tpu_sc as plsc

N = 10000
E = 640000
D = 32
D4 = 4 * D

NC = 2
NS = 16
CHUNK = 40
EDGES_PER_CORE = E // NC
EDGES_PER_TILE = EDGES_PER_CORE // NS
N_CHUNKS = EDGES_PER_TILE // CHUNK
NPAD = 10240
ROWS_PER_TILE = NPAD // NS


NBUF = 4
LOOKAHEAD = 3


def _sc_aggregate(q_flat, ea_flat, src, dst, zeros):
    mesh = plsc.VectorSubcoreMesh(core_axis_name="c", subcore_axis_name="s")

    scratch = [pltpu.VMEM_SHARED((NPAD, D4), jnp.float32)]
    scratch += [pltpu.VMEM((CHUNK, D4), jnp.float32)] * NBUF
    scratch += [pltpu.VMEM((CHUNK, D4), jnp.float32)] * NBUF
    scratch += [pltpu.VMEM((CHUNK,), jnp.int32)] * NBUF
    scratch += [pltpu.VMEM((CHUNK,), jnp.int32)] * NBUF
    scratch += [pltpu.SemaphoreType.DMA] * (6 * NBUF)

    @functools.partial(
        pl.kernel,
        out_type=jax.ShapeDtypeStruct((NC, NPAD, D4), jnp.float32),
        mesh=mesh,
        scratch_types=scratch,
    )
    def k(q_hbm, ea_hbm, src_hbm, dst_hbm, z_hbm, out_hbm,
          accum, *bufs_and_sems):
        ea_v = bufs_and_sems[0 * NBUF:1 * NBUF]
        q_v = bufs_and_sems[1 * NBUF:2 * NBUF]
        src_v = bufs_and_sems[2 * NBUF:3 * NBUF]
        dst_v = bufs_and_sems[3 * NBUF:4 * NBUF]
        sem_e = bufs_and_sems[4 * NBUF:5 * NBUF]
        sem_q = bufs_and_sems[5 * NBUF:6 * NBUF]
        sem_se = bufs_and_sems[6 * NBUF:7 * NBUF]
        sem_sq = bufs_and_sems[7 * NBUF:8 * NBUF]
        sem_is = bufs_and_sems[8 * NBUF:9 * NBUF]
        sem_id = bufs_and_sems[9 * NBUF:10 * NBUF]

        c = lax.axis_index("c")
        s = lax.axis_index("s")
        pltpu.sync_copy(z_hbm, accum.at[pl.ds(s * ROWS_PER_TILE, ROWS_PER_TILE)])
        plsc.subcore_barrier()

        base0 = c * EDGES_PER_CORE + s * EDGES_PER_TILE

        def start_idx(g, par):
            pltpu.async_copy(src_hbm.at[pl.ds(base0 + g * CHUNK, CHUNK)],
                             src_v[par], sem_is[par])
            pltpu.async_copy(dst_hbm.at[pl.ds(base0 + g * CHUNK, CHUNK)],
                             dst_v[par], sem_id[par])

        def wait_idx(par):
            pltpu.make_async_copy(src_hbm.at[pl.ds(0, CHUNK)], src_v[par],
                                  sem_is[par]).wait()
            pltpu.make_async_copy(dst_hbm.at[pl.ds(0, CHUNK)], dst_v[par],
                                  sem_id[par]).wait()

        def start_data(g, par):
            pltpu.async_copy(ea_hbm.at[pl.ds(base0 + g * CHUNK, CHUNK)],
                             ea_v[par], sem_e[par])
            pltpu.async_copy(q_hbm.at[src_v[par]], q_v[par], sem_q[par])

        def finish(g, par):
            pltpu.make_async_copy(ea_hbm.at[pl.ds(0, CHUNK)], ea_v[par],
                                  sem_e[par]).wait()
            pltpu.make_async_copy(ea_hbm.at[pl.ds(0, CHUNK)], q_v[par],
                                  sem_q[par]).wait()
            ce = pltpu.async_copy(ea_v[par], accum.at[dst_v[par]],
                                  sem_se[par], add=True)
            cq = pltpu.async_copy(q_v[par], accum.at[dst_v[par]],
                                  sem_sq[par], add=True)
            ce.wait()
            cq.wait()

        for j in range(NBUF):
            start_idx(j, j)
        for j in range(LOOKAHEAD):
            wait_idx(j)
            start_data(j, j)

        def body(p, carry):
            for j in range(NBUF):
                g = p * NBUF + j
                ga = g + LOOKAHEAD
                pa = (j + LOOKAHEAD) % NBUF

                @pl.when(ga < N_CHUNKS)
                def _():
                    wait_idx(pa)
                    start_data(ga, pa)

                finish(g, j)

                @pl.when(g + NBUF < N_CHUNKS)
                def _():
                    start_idx(g + NBUF, j)
            return carry

        lax.fori_loop(0, N_CHUNKS // NBUF, body, 0)

        plsc.subcore_barrier()
        pltpu.sync_copy(accum.at[pl.ds(s * ROWS_PER_TILE, ROWS_PER_TILE)],
                        out_hbm.at[c, pl.ds(s * ROWS_PER_TILE, ROWS_PER_TILE)])

    return k(q_flat, ea_flat, src, dst, zeros)


def _tc_transform(p, q_flat, h, b_flat):
    blk = 2000

    def body(p_ref, q_ref, h_ref, b_ref, o_ref):
        acc = p_ref[0] + p_ref[1]
        o_ref[...] = (jnp.dot(acc, h_ref[...], preferred_element_type=jnp.float32)
                      + b_ref[...] + q_ref[...])

    return pl.pallas_call(
        body,
        grid=(N // blk,),
        in_specs=[
            pl.BlockSpec((NC, blk, D4), lambda i: (0, i, 0)),
            pl.BlockSpec((blk, D4), lambda i: (i, 0)),
            pl.BlockSpec((D4, D4), lambda i: (0, 0)),
            pl.BlockSpec((1, D4), lambda i: (0, 0)),
        ],
        out_specs=pl.BlockSpec((blk, D4), lambda i: (i, 0)),
        out_shape=jax.ShapeDtypeStruct((N, D4), jnp.float32),
    )(p, q_flat, h, b_flat)


def kernel(q, edge_index, edge_attr, W_r, W_i, W_j, W_k, b):
    q_flat = q.reshape(N, D4)
    ea_flat = edge_attr.reshape(E, D4)
    src = edge_index[0].astype(jnp.int32)
    dst = edge_index[1].astype(jnp.int32)
    zeros = jnp.zeros((ROWS_PER_TILE, D4), jnp.float32)

    p = _sc_aggregate(q_flat, ea_flat, src, dst, zeros)

    h = jnp.concatenate([
        jnp.concatenate([W_r, W_i, W_j, W_k], axis=1),
        jnp.concatenate([-W_i, W_r, -W_k, W_j], axis=1),
        jnp.concatenate([-W_j, W_k, W_r, -W_i], axis=1),
        jnp.concatenate([-W_k, -W_j, W_i, W_r], axis=1),
    ], axis=0)
    b_flat = b.reshape(1, D4)

    out = _tc_transform(p, q_flat, h, b_flat)
    return out.reshape(N, 4, D)


# speedup vs baseline: 112.7803x; 1.1922x over previous
import functools

import jax
import jax.numpy as jnp
from jax import lax
from jax.experimental import pallas as pl
from jax.experimental.pallas import tpu as pltpu
from jax.experimental.pallas import tpu_sc as plsc

N = 10000
E = 640000
D = 32
D4 = 4 * D

NC = 2
NS = 16
CHUNK = 40
EDGES_PER_CORE = E // NC
EDGES_PER_TILE = EDGES_PER_CORE // NS
N_CHUNKS = EDGES_PER_TILE // CHUNK
NPAD = 10240
ROWS_PER_TILE = NPAD // NS

NBUF = 4
LOOKAHEAD = 3

_MESH = plsc.VectorSubcoreMesh(core_axis_name="c", subcore_axis_name="s")


def _sc_aggregate_q(q_flat, src, dst, zeros):
    scratch = [pltpu.VMEM_SHARED((NPAD, D4), jnp.float32)]
    scratch += [pltpu.VMEM((CHUNK, D4), jnp.float32)] * NBUF
    scratch += [pltpu.VMEM((CHUNK,), jnp.int32)] * NBUF
    scratch += [pltpu.VMEM((CHUNK,), jnp.int32)] * NBUF
    scratch += [pltpu.SemaphoreType.DMA] * (4 * NBUF)

    @functools.partial(
        pl.kernel,
        out_type=jax.ShapeDtypeStruct((NC, NPAD, D4), jnp.float32),
        mesh=_MESH,
        scratch_types=scratch,
    )
    def k(q_hbm, src_hbm, dst_hbm, z_hbm, out_hbm, accum, *bs):
        q_v = bs[0 * NBUF:1 * NBUF]
        src_v = bs[1 * NBUF:2 * NBUF]
        dst_v = bs[2 * NBUF:3 * NBUF]
        sem_q = bs[3 * NBUF:4 * NBUF]
        sem_s = bs[4 * NBUF:5 * NBUF]
        sem_is = bs[5 * NBUF:6 * NBUF]
        sem_id = bs[6 * NBUF:7 * NBUF]

        c = lax.axis_index("c")
        s = lax.axis_index("s")
        pltpu.sync_copy(z_hbm, accum.at[pl.ds(s * ROWS_PER_TILE, ROWS_PER_TILE)])
        plsc.subcore_barrier()

        base0 = c * EDGES_PER_CORE + s * EDGES_PER_TILE

        def start_idx(g, par):
            pltpu.async_copy(src_hbm.at[pl.ds(base0 + g * CHUNK, CHUNK)],
                             src_v[par], sem_is[par])
            pltpu.async_copy(dst_hbm.at[pl.ds(base0 + g * CHUNK, CHUNK)],
                             dst_v[par], sem_id[par])

        def wait_idx(par):
            pltpu.make_async_copy(src_hbm.at[pl.ds(0, CHUNK)], src_v[par],
                                  sem_is[par]).wait()
            pltpu.make_async_copy(dst_hbm.at[pl.ds(0, CHUNK)], dst_v[par],
                                  sem_id[par]).wait()

        def start_data(g, par):
            pltpu.async_copy(q_hbm.at[src_v[par]], q_v[par], sem_q[par])

        def finish(g, par):
            pltpu.make_async_copy(q_hbm.at[pl.ds(0, CHUNK)], q_v[par],
                                  sem_q[par]).wait()
            pltpu.async_copy(q_v[par], accum.at[dst_v[par]],
                             sem_s[par], add=True).wait()

        for j in range(NBUF):
            start_idx(j, j)
        for j in range(LOOKAHEAD):
            wait_idx(j)
            start_data(j, j)

        def body(p, carry):
            for j in range(NBUF):
                g = p * NBUF + j
                ga = g + LOOKAHEAD
                pa = (j + LOOKAHEAD) % NBUF

                @pl.when(ga < N_CHUNKS)
                def _():
                    wait_idx(pa)
                    start_data(ga, pa)

                finish(g, j)

                @pl.when(g + NBUF < N_CHUNKS)
                def _():
                    start_idx(g + NBUF, j)
            return carry

        lax.fori_loop(0, N_CHUNKS // NBUF, body, 0)

        plsc.subcore_barrier()
        pltpu.sync_copy(accum.at[pl.ds(s * ROWS_PER_TILE, ROWS_PER_TILE)],
                        out_hbm.at[c, pl.ds(s * ROWS_PER_TILE, ROWS_PER_TILE)])

    return k(q_flat, src, dst, zeros)


def _sc_aggregate_ea(ea_flat, dst, zeros):
    scratch = [pltpu.VMEM_SHARED((NPAD, D4), jnp.float32)]
    scratch += [pltpu.VMEM((CHUNK, D4), jnp.float32)] * NBUF
    scratch += [pltpu.VMEM((CHUNK,), jnp.int32)] * NBUF
    scratch += [pltpu.SemaphoreType.DMA] * (3 * NBUF)

    @functools.partial(
        pl.kernel,
        out_type=jax.ShapeDtypeStruct((NC, NPAD, D4), jnp.float32),
        mesh=_MESH,
        scratch_types=scratch,
    )
    def k(ea_hbm, dst_hbm, z_hbm, out_hbm, accum, *bs):
        ea_v = bs[0 * NBUF:1 * NBUF]
        dst_v = bs[1 * NBUF:2 * NBUF]
        sem_e = bs[2 * NBUF:3 * NBUF]
        sem_s = bs[3 * NBUF:4 * NBUF]
        sem_id = bs[4 * NBUF:5 * NBUF]

        c = lax.axis_index("c")
        s = lax.axis_index("s")
        pltpu.sync_copy(z_hbm, accum.at[pl.ds(s * ROWS_PER_TILE, ROWS_PER_TILE)])
        plsc.subcore_barrier()

        base0 = c * EDGES_PER_CORE + s * EDGES_PER_TILE

        def start_idx(g, par):
            pltpu.async_copy(dst_hbm.at[pl.ds(base0 + g * CHUNK, CHUNK)],
                             dst_v[par], sem_id[par])

        def wait_idx(par):
            pltpu.make_async_copy(dst_hbm.at[pl.ds(0, CHUNK)], dst_v[par],
                                  sem_id[par]).wait()

        def start_data(g, par):
            pltpu.async_copy(ea_hbm.at[pl.ds(base0 + g * CHUNK, CHUNK)],
                             ea_v[par], sem_e[par])

        def finish(g, par):
            pltpu.make_async_copy(ea_hbm.at[pl.ds(0, CHUNK)], ea_v[par],
                                  sem_e[par]).wait()
            pltpu.async_copy(ea_v[par], accum.at[dst_v[par]],
                             sem_s[par], add=True).wait()

        for j in range(NBUF):
            start_idx(j, j)
        for j in range(LOOKAHEAD):
            start_data(j, j)

        def body(p, carry):
            for j in range(NBUF):
                g = p * NBUF + j
                ga = g + LOOKAHEAD
                pa = (j + LOOKAHEAD) % NBUF

                @pl.when(ga < N_CHUNKS)
                def _():
                    start_data(ga, pa)

                wait_idx(j)
                finish(g, j)

                @pl.when(g + NBUF < N_CHUNKS)
                def _():
                    start_idx(g + NBUF, j)
            return carry

        lax.fori_loop(0, N_CHUNKS // NBUF, body, 0)

        plsc.subcore_barrier()
        pltpu.sync_copy(accum.at[pl.ds(s * ROWS_PER_TILE, ROWS_PER_TILE)],
                        out_hbm.at[c, pl.ds(s * ROWS_PER_TILE, ROWS_PER_TILE)])

    return k(ea_flat, dst, zeros)


def _tc_transform(pa, pb, q_flat, h, b_flat):
    blk = 2000

    def body(pa_ref, pb_ref, q_ref, h_ref, b_ref, o_ref):
        acc = (pa_ref[0] + pa_ref[1]) + (pb_ref[0] + pb_ref[1])
        o_ref[...] = (jnp.dot(acc, h_ref[...], preferred_element_type=jnp.float32)
                      + b_ref[...] + q_ref[...])

    return pl.pallas_call(
        body,
        grid=(N // blk,),
        in_specs=[
            pl.BlockSpec((NC, blk, D4), lambda i: (0, i, 0)),
            pl.BlockSpec((NC, blk, D4), lambda i: (0, i, 0)),
            pl.BlockSpec((blk, D4), lambda i: (i, 0)),
            pl.BlockSpec((D4, D4), lambda i: (0, 0)),
            pl.BlockSpec((1, D4), lambda i: (0, 0)),
        ],
        out_specs=pl.BlockSpec((blk, D4), lambda i: (i, 0)),
        out_shape=jax.ShapeDtypeStruct((N, D4), jnp.float32),
    )(pa, pb, q_flat, h, b_flat)


def kernel(q, edge_index, edge_attr, W_r, W_i, W_j, W_k, b):
    q_flat = q.reshape(N, D4)
    ea_flat = edge_attr.reshape(E, D4)
    src = edge_index[0].astype(jnp.int32)
    dst = edge_index[1].astype(jnp.int32)
    zeros = jnp.zeros((ROWS_PER_TILE, D4), jnp.float32)

    pa = _sc_aggregate_q(q_flat, src, dst, zeros)
    pb = _sc_aggregate_ea(ea_flat, dst, zeros)

    h = jnp.concatenate([
        jnp.concatenate([W_r, W_i, W_j, W_k], axis=1),
        jnp.concatenate([-W_i, W_r, -W_k, W_j], axis=1),
        jnp.concatenate([-W_j, W_k, W_r, -W_i], axis=1),
        jnp.concatenate([-W_k, -W_j, W_i, W_r], axis=1),
    ], axis=0)
    b_flat = b.reshape(1, D4)

    out = _tc_transform(pa, pb, q_flat, h, b_flat)
    return out.reshape(N, 4, D)
